# baseline (speedup 1.0000x reference)
import jax
import jax.numpy as jnp
from jax import lax
from jax.experimental import pallas as pl
from jax.experimental.pallas import tpu as pltpu
from jax.experimental.pallas import tpu_sc as plsc

N = 16384
D_ATOM = 16
D_CHARGE = 32
D_NUM = 16
D_OUT = 64
V_ATOM = 1000000
V_LAST = (V_ATOM // 128) - 1
A_TAIL = (V_LAST + 1) * 128

NC = 2
NS = 16
NW = NC * NS
B_W = N // NW
HG = 8
NHG = B_W // HG


def _body(ef_t, wa_t, wa_tail, wc, at_hbm, cs_hbm, out_t,
          idx_a, idx_c, slabs_a, slabs_c, tail_v, out_v,
          sem_i, sem_a0, sem_c0, sem_a1, sem_c1):
    wid = lax.axis_index("s") * NC + lax.axis_index("c")
    base = wid * B_W
    sems = ((sem_a0, sem_c0), (sem_a1, sem_c1))
    rows16 = lax.iota(jnp.int32, 16)

    cp_ia = pltpu.make_async_copy(at_hbm.at[pl.ds(base, B_W)], idx_a, sem_i)
    cp_ic = pltpu.make_async_copy(cs_hbm.at[pl.ds(base, B_W)], idx_c, sem_i)
    cp_t = pltpu.make_async_copy(wa_tail, tail_v, sem_i)
    cp_ia.start()
    cp_ic.start()
    cp_t.start()
    cp_ia.wait()
    cp_ic.wait()
    cp_t.wait()
    pltpu.sync_copy(ef_t.at[:, pl.ds(base, B_W)],
                    out_v.at[pl.ds(D_ATOM + D_CHARGE, D_NUM), :])

    def issue_half_at(vec_a, vec_c, lane0, b):
        sa, sc = sems[b]
        for n in range(HG):
            v = vec_a[lane0 + n]
            j = jnp.minimum(v >> 7, V_LAST).astype(jnp.int32)
            pltpu.make_async_copy(
                wa_t.at[:, pl.ds(j * 128, 128)], slabs_a.at[b, n], sa).start()
            w = vec_c[lane0 + n]
            k = (w >> 3).astype(jnp.int32)
            pltpu.make_async_copy(
                wc.at[pl.ds(k * 8, 8), :], slabs_c.at[b, n], sc).start()

    def wait_half(b):
        sa, sc = sems[b]
        for n in range(HG):
            pltpu.make_async_copy(
                wa_t.at[:, pl.ds(0, 128)], slabs_a.at[b, n], sa).wait()
            pltpu.make_async_copy(
                wc.at[pl.ds(0, 8), :], slabs_c.at[b, n], sc).wait()

    def extract_half(hg, vec_a, vec_c, lane0, b):
        for n in range(HG):
            v = vec_a[lane0 + n]
            j = jnp.minimum(v >> 7, V_LAST).astype(jnp.int32)
            l = jnp.minimum(v - (j << 7), 127).astype(jnp.int32)
            colv = jnp.full((16,), hg * HG + n, jnp.int32)
            a_main = plsc.load_gather(
                slabs_a.at[b, n], [rows16, jnp.full((16,), l, jnp.int32)])
            t = jnp.clip(v - A_TAIL, 0, 63).astype(jnp.int32)
            a_tail = plsc.load_gather(
                tail_v, [rows16, jnp.full((16,), t, jnp.int32)])
            a = jnp.where(v >= A_TAIL, a_tail, a_main)
            plsc.store_scatter(out_v, [rows16, colv], a)
            w = vec_c[lane0 + n]
            k = (w >> 3).astype(jnp.int32)
            r = (w - (k << 3)).astype(jnp.int32)
            c0 = slabs_c[b, n, r, pl.ds(0, 16)]
            c1 = slabs_c[b, n, r, pl.ds(16, 16)]
            plsc.store_scatter(out_v, [rows16 + D_ATOM, colv], c0)
            plsc.store_scatter(out_v, [rows16 + D_ATOM + 16, colv], c1)

    va0 = idx_a[pl.ds(0, 16)]
    vc0 = idx_c[pl.ds(0, 16)]
    issue_half_at(va0, vc0, 0, 0)
    issue_half_at(va0, vc0, HG, 1)

    def step(i, _):
        hg0 = 2 * i
        va = idx_a[pl.ds(i * 16, 16)]
        vc = idx_c[pl.ds(i * 16, 16)]

        wait_half(0)
        extract_half(hg0, va, vc, 0, 0)

        @pl.when(i + 1 < NHG // 2)
        def _issue0():
            van = idx_a[pl.ds((i + 1) * 16, 16)]
            vcn = idx_c[pl.ds((i + 1) * 16, 16)]
            issue_half_at(van, vcn, 0, 0)

        wait_half(1)
        extract_half(hg0 + 1, va, vc, HG, 1)

        @pl.when(i + 1 < NHG // 2)
        def _issue1():
            van = idx_a[pl.ds((i + 1) * 16, 16)]
            vcn = idx_c[pl.ds((i + 1) * 16, 16)]
            issue_half_at(van, vcn, HG, 1)

        return _

    lax.fori_loop(0, NHG // 2, step, 0)

    pltpu.sync_copy(out_v, out_t.at[:, pl.ds(base, B_W)])


@jax.jit
def _lookup(ef_t, wa_t, wa_tail, wc, at, cs):
    mesh = plsc.VectorSubcoreMesh(core_axis_name="c", subcore_axis_name="s")
    return pl.kernel(
        _body,
        out_type=jax.ShapeDtypeStruct((D_OUT, N), jnp.float32),
        mesh=mesh,
        scratch_types=[
            pltpu.VMEM((B_W,), jnp.int32),
            pltpu.VMEM((B_W,), jnp.int32),
            pltpu.VMEM((2, HG, D_ATOM, 128), jnp.float32),
            pltpu.VMEM((2, HG, 8, D_CHARGE), jnp.float32),
            pltpu.VMEM((D_ATOM, 64), jnp.float32),
            pltpu.VMEM((D_OUT, B_W), jnp.float32),
            pltpu.SemaphoreType.DMA,
            pltpu.SemaphoreType.DMA,
            pltpu.SemaphoreType.DMA,
            pltpu.SemaphoreType.DMA,
            pltpu.SemaphoreType.DMA,
        ],
        compiler_params=pltpu.CompilerParams(needs_layout_passes=False),
    )(ef_t, wa_t, wa_tail, wc, at, cs)


def kernel(pos, extra_feat, W_atom, W_charge, atom_type, charge_state):
    out_t = _lookup(extra_feat.T, W_atom.T, W_atom[A_TAIL:].T, W_charge,
                    atom_type, charge_state)
    return out_t.T.astype(pos.dtype)


# speedup vs baseline: 1.0025x; 1.0025x over previous
import jax
import jax.numpy as jnp
from jax import lax
from jax.experimental import pallas as pl
from jax.experimental.pallas import tpu as pltpu
from jax.experimental.pallas import tpu_sc as plsc

N = 16384
D_ATOM = 16
D_CHARGE = 32
D_NUM = 16
D_OUT = 64
V_ATOM = 1000000
V_LAST = (V_ATOM // 128) - 1
A_TAIL = (V_LAST + 1) * 128

NC = 2
NS = 16
NW = NC * NS
B_W = N // NW
HG = 8
NHG = B_W // HG


def _body(ef_t, wa_t, wa_tail, wc, at_hbm, cs_hbm, out_t,
          idx_a, idx_c, slabs_a, slabs_c, tail_v, out_v,
          sem_i, sem_a0, sem_c0, sem_a1, sem_c1):
    wid = lax.axis_index("s") * NC + lax.axis_index("c")
    base = wid * B_W
    sems = ((sem_a0, sem_c0), (sem_a1, sem_c1))
    rows16 = lax.iota(jnp.int32, 16)

    cp_ia = pltpu.make_async_copy(at_hbm.at[pl.ds(base, B_W)], idx_a, sem_i)
    cp_ic = pltpu.make_async_copy(cs_hbm.at[pl.ds(base, B_W)], idx_c, sem_i)
    cp_t = pltpu.make_async_copy(wa_tail, tail_v, sem_i)
    cp_ia.start()
    cp_ic.start()
    cp_t.start()
    cp_ia.wait()
    cp_ic.wait()
    cp_t.wait()
    pltpu.sync_copy(ef_t.at[:, pl.ds(base, B_W)],
                    out_v.at[pl.ds(D_ATOM + D_CHARGE, D_NUM), :])

    def issue_half_at(vec_a, vec_c, lane0, b):
        sa, sc = sems[b]
        for n in range(HG):
            v = vec_a[lane0 + n]
            j = jnp.minimum(v >> 7, V_LAST).astype(jnp.int32)
            pltpu.make_async_copy(
                wa_t.at[:, pl.ds(j * 128, 128)], slabs_a.at[b, n], sa).start()
            w = vec_c[lane0 + n]
            k = (w >> 3).astype(jnp.int32)
            pltpu.make_async_copy(
                wc.at[pl.ds(k * 8, 8), :], slabs_c.at[b, n], sc).start()

    def wait_half(b):
        sa, sc = sems[b]
        for n in range(HG):
            pltpu.make_async_copy(
                wa_t.at[:, pl.ds(0, 128)], slabs_a.at[b, n], sa).wait()
            pltpu.make_async_copy(
                wc.at[pl.ds(0, 8), :], slabs_c.at[b, n], sc).wait()

    def extract_half(hg, vec_a, vec_c, lane0, b):
        for n in range(HG):
            v = vec_a[lane0 + n]
            j = jnp.minimum(v >> 7, V_LAST).astype(jnp.int32)
            l = jnp.minimum(v - (j << 7), 127).astype(jnp.int32)
            colv = jnp.full((16,), hg * HG + n, jnp.int32)
            a_main = plsc.load_gather(
                slabs_a.at[b, n], [rows16, jnp.full((16,), l, jnp.int32)])
            t = jnp.clip(v - A_TAIL, 0, 63).astype(jnp.int32)
            a_tail = plsc.load_gather(
                tail_v, [rows16, jnp.full((16,), t, jnp.int32)])
            a = jnp.where(v >= A_TAIL, a_tail, a_main)
            plsc.store_scatter(out_v, [rows16, colv], a)
            w = vec_c[lane0 + n]
            k = (w >> 3).astype(jnp.int32)
            r = (w - (k << 3)).astype(jnp.int32)
            c0 = slabs_c[b, n, r, pl.ds(0, 16)]
            c1 = slabs_c[b, n, r, pl.ds(16, 16)]
            plsc.store_scatter(out_v, [rows16 + D_ATOM, colv], c0)
            plsc.store_scatter(out_v, [rows16 + D_ATOM + 16, colv], c1)

    va0 = idx_a[pl.ds(0, 16)]
    vc0 = idx_c[pl.ds(0, 16)]
    issue_half_at(va0, vc0, 0, 0)
    issue_half_at(va0, vc0, HG, 1)

    def step(i, _):
        hg0 = 2 * i
        va = idx_a[pl.ds(i * 16, 16)]
        vc = idx_c[pl.ds(i * 16, 16)]

        wait_half(0)
        extract_half(hg0, va, vc, 0, 0)

        @pl.when(i + 1 < NHG // 2)
        def _issue0():
            van = idx_a[pl.ds((i + 1) * 16, 16)]
            vcn = idx_c[pl.ds((i + 1) * 16, 16)]
            issue_half_at(van, vcn, 0, 0)

        wait_half(1)
        extract_half(hg0 + 1, va, vc, HG, 1)

        @pl.when(i + 1 < NHG // 2)
        def _issue1():
            van = idx_a[pl.ds((i + 1) * 16, 16)]
            vcn = idx_c[pl.ds((i + 1) * 16, 16)]
            issue_half_at(van, vcn, HG, 1)

        return _

    lax.fori_loop(0, NHG // 2, step, 0)

    pltpu.sync_copy(out_v, out_t.at[:, pl.ds(base, B_W)])


@jax.jit
def _lookup(ef_t, wa_t, wa_tail, wc, at, cs):
    mesh = plsc.VectorSubcoreMesh(core_axis_name="c", subcore_axis_name="s")
    return pl.kernel(
        _body,
        out_type=jax.ShapeDtypeStruct((D_OUT, N), jnp.float32),
        mesh=mesh,
        scratch_types=[
            pltpu.VMEM((B_W,), jnp.int32),
            pltpu.VMEM((B_W,), jnp.int32),
            pltpu.VMEM((2, HG, D_ATOM, 128), jnp.float32),
            pltpu.VMEM((2, HG, 8, D_CHARGE), jnp.float32),
            pltpu.VMEM((D_ATOM, 64), jnp.float32),
            pltpu.VMEM((D_OUT, B_W), jnp.float32),
            pltpu.SemaphoreType.DMA,
            pltpu.SemaphoreType.DMA,
            pltpu.SemaphoreType.DMA,
            pltpu.SemaphoreType.DMA,
            pltpu.SemaphoreType.DMA,
        ],
        compiler_params=pltpu.CompilerParams(
            needs_layout_passes=False, skip_device_barrier=True),
    )(ef_t, wa_t, wa_tail, wc, at, cs)


def kernel(pos, extra_feat, W_atom, W_charge, atom_type, charge_state):
    out_t = _lookup(extra_feat.T, W_atom.T, W_atom[A_TAIL:].T, W_charge,
                    atom_type, charge_state)
    return out_t.T.astype(pos.dtype)
